# Initial kernel scaffold; baseline (speedup 1.0000x reference)
import functools

import jax
import jax.numpy as jnp
from jax import lax
from jax.experimental import pallas as pl
from jax.experimental.pallas import tpu as pltpu
from jax.experimental.pallas import tpu_sc as plsc

_VOCAB = 100000
_D = 128
_POS_DIM = 5
_POS_VOCAB = 201
_B = 4096
_L = 200
_NC = 2
_NS = 16
_NW = _NC * _NS
_BPW = _B // _NW


def _ea_body(X, XP1, XP2, XE1, XE2, table, p1w, p2w, xp_out, xe_out,
             idx_lo, idx_hi, p1_idx, p2_idx, e1_idx, e2_idx,
             word_rows, e1_rows, e2_rows, e12, pos_stage, p1_v, p2_v, sem):
    wid = lax.axis_index("s") * _NC + lax.axis_index("c")
    b0 = wid * _BPW

    pltpu.sync_copy(XE1.at[pl.ds(b0, _BPW)], e1_idx)
    pltpu.sync_copy(XE2.at[pl.ds(b0, _BPW)], e2_idx)
    pltpu.async_copy(table.at[e1_idx], e1_rows, sem).wait()
    pltpu.async_copy(table.at[e2_idx], e2_rows, sem).wait()
    pltpu.sync_copy(p1w, p1_v)
    pltpu.sync_copy(p2w, p2_v)

    lane = lax.iota(jnp.int32, (16,))

    def body(i, carry):
        b = b0 + i
        pltpu.sync_copy(X.at[b, pl.ds(0, 104)], idx_lo)
        pltpu.sync_copy(X.at[b, pl.ds(104, 96)], idx_hi.at[pl.ds(0, 96)])
        tail = idx_hi[pl.ds(88, 16)]
        idx_hi[pl.ds(88, 16)] = jnp.where(lane < 8, tail, 0)

        pltpu.async_copy(table.at[idx_lo], word_rows.at[pl.ds(0, 104)], sem).wait()
        pltpu.async_copy(table.at[idx_hi], word_rows.at[pl.ds(104, 104)], sem).wait()

        pltpu.sync_copy(word_rows.at[pl.ds(0, _L)], xp_out.at[b, :, pl.ds(0, _D)])
        pltpu.sync_copy(word_rows.at[pl.ds(0, _L)], xe_out.at[b, :, pl.ds(0, _D)])

        pltpu.sync_copy(XP1.at[b], p1_idx.at[pl.ds(0, _L)])
        pltpu.sync_copy(XP2.at[b], p2_idx.at[pl.ds(0, _L)])

        def pos_body(t, c):
            rows = t * 16 + lane
            msk = rows < _L
            for tbl, pidx, cbase in ((p1_v, p1_idx, 0), (p2_v, p2_idx, _POS_DIM)):
                iv = pidx[pl.ds(t * 16, 16)]
                iv = jnp.where(msk, iv, 0)
                for j in range(_POS_DIM):
                    colv = jnp.full((16,), j, jnp.int32)
                    vals = plsc.load_gather(tbl, [iv, colv])
                    plsc.store_scatter(pos_stage, [rows, colv + cbase], vals, msk)
            return c

        lax.fori_loop(0, 13, pos_body, 0)
        pltpu.sync_copy(pos_stage, xp_out.at[b, :, pl.ds(_D, 2 * _POS_DIM)])

        evs = ([e1_rows[i, pl.ds(16 * v, 16)] for v in range(8)] +
               [e2_rows[i, pl.ds(16 * v, 16)] for v in range(8)])

        def brow(r, c):
            for v in range(16):
                e12[r, pl.ds(16 * v, 16)] = evs[v]
            return c

        lax.fori_loop(0, _L, brow, 0)
        pltpu.sync_copy(e12, xe_out.at[b, :, pl.ds(_D, 2 * _D)])
        return carry

    lax.fori_loop(0, _BPW, body, 0)


@jax.jit
def _run(X, XP1, XP2, XE1, XE2, table, p1w, p2w):
    mesh = plsc.VectorSubcoreMesh(core_axis_name="c", subcore_axis_name="s")
    f = pl.kernel(
        _ea_body,
        mesh=mesh,
        out_type=(
            jax.ShapeDtypeStruct((_B, _L, _D + 2 * _POS_DIM), jnp.float32),
            jax.ShapeDtypeStruct((_B, _L, 3 * _D), jnp.float32),
        ),
        scratch_types=[
            pltpu.VMEM((104,), jnp.int32),
            pltpu.VMEM((104,), jnp.int32),
            pltpu.VMEM((208,), jnp.int32),
            pltpu.VMEM((208,), jnp.int32),
            pltpu.VMEM((_BPW,), jnp.int32),
            pltpu.VMEM((_BPW,), jnp.int32),
            pltpu.VMEM((208, _D), jnp.float32),
            pltpu.VMEM((_BPW, _D), jnp.float32),
            pltpu.VMEM((_BPW, _D), jnp.float32),
            pltpu.VMEM((_L, 2 * _D), jnp.float32),
            pltpu.VMEM((_L, 2 * _POS_DIM), jnp.float32),
            pltpu.VMEM((_POS_VOCAB, _POS_DIM), jnp.float32),
            pltpu.VMEM((_POS_VOCAB, _POS_DIM), jnp.float32),
            pltpu.SemaphoreType.DMA,
        ],
    )
    return f(X, XP1, XP2, XE1, XE2, table, p1w, p2w)


def kernel(X, X_Pos1, X_Pos2, X_Ent1, X_Ent2, word_embedding, pos1_weight, pos2_weight):
    return _run(X, X_Pos1, X_Pos2, X_Ent1, X_Ent2, word_embedding,
                pos1_weight, pos2_weight)


# speedup vs baseline: 2.7042x; 2.7042x over previous
import functools

import jax
import jax.numpy as jnp
from jax import lax
from jax.experimental import pallas as pl
from jax.experimental.pallas import tpu as pltpu
from jax.experimental.pallas import tpu_sc as plsc

_VOCAB = 100000
_D = 128
_POS_DIM = 5
_POS_VOCAB = 201
_B = 4096
_L = 200
_NC = 2
_NS = 16
_NW = _NC * _NS
_BPW = _B // _NW


def _ea_body(X, XP1, XP2, XE1, XE2, table, p1w, p2w, xp_out, xe_out,
             idx_lo, idx_hi, p1_idx, p2_idx, e1_idx, e2_idx,
             word_rows, e1_rows, e2_rows, e12, pos_stage, p1_v, p2_v, sem):
    wid = lax.axis_index("s") * _NC + lax.axis_index("c")
    b0 = wid * _BPW

    pltpu.sync_copy(p1w, p1_v)
    pltpu.sync_copy(p2w, p2_v)

    lane = lax.iota(jnp.int32, 16)

    def group(g, carry_g):
        gb0 = b0 + g * 8
        pltpu.sync_copy(XE1.at[pl.ds(gb0, 8)], e1_idx)
        pltpu.sync_copy(XE2.at[pl.ds(gb0, 8)], e2_idx)
        pltpu.async_copy(table.at[e1_idx], e1_rows, sem).wait()
        pltpu.async_copy(table.at[e2_idx], e2_rows, sem).wait()
        return _inner(gb0, carry_g)

    def _inner(gb0, carry_g):
        lax.fori_loop(0, 8, lambda i, c: body(gb0, i, c), 0)
        return carry_g

    def body(gb0, i, carry):
        b = gb0 + i
        pltpu.sync_copy(X.at[pl.ds(b * _L, 104)], idx_lo)
        pltpu.sync_copy(X.at[pl.ds(b * _L + 104, 96)], idx_hi.at[pl.ds(0, 96)])
        tail = idx_hi[pl.ds(88, 16)]
        idx_hi[pl.ds(88, 16)] = jnp.where(lane < 8, tail, 0)

        pltpu.async_copy(table.at[idx_lo], word_rows.at[pl.ds(0, 104)], sem).wait()
        pltpu.async_copy(table.at[idx_hi], word_rows.at[pl.ds(104, 104)], sem).wait()

        pltpu.sync_copy(word_rows.at[pl.ds(0, _L)], xp_out.at[b, :, pl.ds(0, _D)])
        pltpu.sync_copy(word_rows.at[pl.ds(0, _L)], xe_out.at[b, :, pl.ds(0, _D)])

        pltpu.sync_copy(XP1.at[pl.ds(b * _L, _L)], p1_idx.at[pl.ds(0, _L)])
        pltpu.sync_copy(XP2.at[pl.ds(b * _L, _L)], p2_idx.at[pl.ds(0, _L)])

        def pos_body(t, c):
            rows = t * 16 + lane
            msk = rows < _L
            for tbl, pidx, cbase in ((p1_v, p1_idx, 0), (p2_v, p2_idx, _POS_DIM)):
                iv = pidx[pl.ds(t * 16, 16)]
                iv = jnp.where(msk, iv * _POS_DIM, 0)
                for j in range(_POS_DIM):
                    colv = jnp.full((16,), j, jnp.int32)
                    vals = plsc.load_gather(tbl, [iv + j])
                    plsc.store_scatter(pos_stage, [rows, colv + cbase], vals, mask=msk)
            return c

        lax.fori_loop(0, 13, pos_body, 0)
        pltpu.sync_copy(pos_stage, xp_out.at[b, :, pl.ds(_D, 2 * _POS_DIM)])

        evs = ([e1_rows[i, pl.ds(16 * v, 16)] for v in range(8)] +
               [e2_rows[i, pl.ds(16 * v, 16)] for v in range(8)])

        def brow(r, c):
            for v in range(16):
                e12[r, pl.ds(16 * v, 16)] = evs[v]
            return c

        lax.fori_loop(0, _L, brow, 0)
        pltpu.sync_copy(e12, xe_out.at[b, :, pl.ds(_D, 2 * _D)])
        return carry

    lax.fori_loop(0, _BPW // 8, group, 0)


@jax.jit
def _run(X, XP1, XP2, XE1, XE2, table, p1w, p2w):
    mesh = plsc.VectorSubcoreMesh(core_axis_name="c", subcore_axis_name="s")
    f = pl.kernel(
        _ea_body,
        mesh=mesh,
        compiler_params=pltpu.CompilerParams(needs_layout_passes=False),
        out_type=(
            jax.ShapeDtypeStruct((_B, _L, _D + 2 * _POS_DIM), jnp.float32),
            jax.ShapeDtypeStruct((_B, _L, 3 * _D), jnp.float32),
        ),
        scratch_types=[
            pltpu.VMEM((104,), jnp.int32),
            pltpu.VMEM((104,), jnp.int32),
            pltpu.VMEM((208,), jnp.int32),
            pltpu.VMEM((208,), jnp.int32),
            pltpu.VMEM((8,), jnp.int32),
            pltpu.VMEM((8,), jnp.int32),
            pltpu.VMEM((208, _D), jnp.float32),
            pltpu.VMEM((8, _D), jnp.float32),
            pltpu.VMEM((8, _D), jnp.float32),
            pltpu.VMEM((_L, 2 * _D), jnp.float32),
            pltpu.VMEM((_L, 2 * _POS_DIM), jnp.float32),
            pltpu.VMEM((1024,), jnp.float32),
            pltpu.VMEM((1024,), jnp.float32),
            pltpu.SemaphoreType.DMA,
        ],
    )
    return f(X, XP1, XP2, XE1, XE2, table, p1w, p2w)


def kernel(X, X_Pos1, X_Pos2, X_Ent1, X_Ent2, word_embedding, pos1_weight, pos2_weight):
    p1f = jnp.pad(pos1_weight.reshape(-1), (0, 1024 - _POS_VOCAB * _POS_DIM))
    p2f = jnp.pad(pos2_weight.reshape(-1), (0, 1024 - _POS_VOCAB * _POS_DIM))
    return _run(X.reshape(-1), X_Pos1.reshape(-1), X_Pos2.reshape(-1),
                X_Ent1, X_Ent2, word_embedding, p1f, p2f)


# speedup vs baseline: 6.9414x; 2.5669x over previous
import jax
import jax.numpy as jnp
from jax import lax
from jax.experimental import pallas as pl
from jax.experimental.pallas import tpu as pltpu
from jax.experimental.pallas import tpu_sc as plsc

_VOCAB = 100000
_D = 128
_POS_DIM = 5
_POS_VOCAB = 201
_B = 4096
_L = 200
_NC = 2
_NS = 16
_NW = _NC * _NS
_BPW = _B // _NW
_GRP = 8
_NGRP = _BPW // _GRP
_GTOK = _GRP * _L
_EROWS = 40


def _ea_body(X, XP1, XP2, XE1, XE2, table, p1w, p2w, xp_out, xe_out,
             x_idx0, x_idx1, p1_idx0, p1_idx1, p2_idx0, p2_idx1,
             e_idx0, e_idx1, e_rows0, e_rows1, word0, word1,
             e12a, e12b, pos_sc, p1_v, p2_v,
             sem_g0, sem_g1, sem_w0, sem_w1, sem_o0, sem_o1,
             sem_r0, sem_r1, sem_p):
    sem_g = (sem_g0, sem_g1)
    sem_w = (sem_w0, sem_w1)
    sem_o = (sem_o0, sem_o1)
    sem_r = (sem_r0, sem_r1)
    x_idx = (x_idx0, x_idx1)
    p_idx = ((p1_idx0, p1_idx1), (p2_idx0, p2_idx1))
    e_idx = (e_idx0, e_idx1)
    e_rows = (e_rows0, e_rows1)
    word = (word0, word1)
    e12 = (e12a, e12b)

    wid = lax.axis_index("s") * _NC + lax.axis_index("c")
    b0 = wid * _BPW
    lane = lax.iota(jnp.int32, 16)

    def grp_descs(kg, g):
        tok0 = (b0 + g * _GRP) * _L
        return (
            (table.at[e_idx[kg].at[pl.ds(0, _GRP)]], e_rows[kg].at[pl.ds(0, _GRP)], sem_r[kg]),
            (table.at[e_idx[kg].at[pl.ds(_GRP, _GRP)]], e_rows[kg].at[pl.ds(_GRP, _GRP)], sem_r[kg]),
            (X.at[pl.ds(tok0, _GTOK)], x_idx[kg].at[pl.ds(0, _GTOK)], sem_r[kg]),
            (XP1.at[pl.ds(tok0, _GTOK)], p_idx[0][kg].at[pl.ds(0, _GTOK)], sem_r[kg]),
            (XP2.at[pl.ds(tok0, _GTOK)], p_idx[1][kg].at[pl.ds(0, _GTOK)], sem_r[kg]),
        )

    def grp_drain_descs(kg, g):
        tok0 = (b0 + g * _GRP) * _L
        return (
            (table.at[pl.ds(0, _GRP)], e_rows[kg].at[pl.ds(0, _GRP)], sem_r[kg]),
            (table.at[pl.ds(0, _GRP)], e_rows[kg].at[pl.ds(_GRP, _GRP)], sem_r[kg]),
            (X.at[pl.ds(tok0, _GTOK)], x_idx[kg].at[pl.ds(0, _GTOK)], sem_r[kg]),
            (XP1.at[pl.ds(tok0, _GTOK)], p_idx[0][kg].at[pl.ds(0, _GTOK)], sem_r[kg]),
            (XP2.at[pl.ds(tok0, _GTOK)], p_idx[1][kg].at[pl.ds(0, _GTOK)], sem_r[kg]),
        )

    def issue_grp(kg, g, bg):
        pltpu.sync_copy(XE1.at[pl.ds(bg, _GRP)], e_idx[kg].at[pl.ds(0, _GRP)])
        pltpu.sync_copy(XE2.at[pl.ds(bg, _GRP)], e_idx[kg].at[pl.ds(_GRP, _GRP)])
        for d in grp_descs(kg, g):
            pltpu.async_copy(*d)

    def drain_grp(kg, g):
        for d in grp_drain_descs(kg, g):
            pltpu.make_async_copy(*d).wait()

    def issue_gather(kg, s, p):
        t0 = s * _L
        pltpu.async_copy(table.at[x_idx[kg].at[pl.ds(t0, 104)]],
                         word[p].at[pl.ds(0, 104)], sem_g[p])
        pltpu.async_copy(table.at[x_idx[kg].at[pl.ds(t0 + 104, 96)]],
                         word[p].at[pl.ds(104, 96)], sem_g[p])

    def drain_gather(p):
        pltpu.make_async_copy(table.at[pl.ds(0, 104)],
                              word[p].at[pl.ds(0, 104)], sem_g[p]).wait()
        pltpu.make_async_copy(table.at[pl.ds(0, 96)],
                              word[p].at[pl.ds(104, 96)], sem_g[p]).wait()

    def word_descs(p, b):
        return (
            (word[p], xp_out.at[b, :, pl.ds(0, _D)], sem_w[p]),
            (word[p], xe_out.at[b, :, pl.ds(0, _D)], sem_w[p]),
        )

    def other_descs(p, b):
        ds = []
        for k in range(_L // _EROWS):
            ds.append((e12[p],
                       xe_out.at[b, pl.ds(_EROWS * k, _EROWS), pl.ds(_D, 2 * _D)],
                       sem_o[p]))
        return ds

    def pos_desc(b):
        return (pos_sc, xp_out.at[b, :, pl.ds(_D, 2 * _POS_DIM)], sem_p)

    def issue_scatters(p, b):
        for d in word_descs(p, b) + tuple(other_descs(p, b)) + (pos_desc(b),):
            pltpu.async_copy(*d)

    def drain_word(p, b):
        for d in word_descs(p, b):
            pltpu.make_async_copy(*d).wait()

    def drain_other(p, b):
        for d in other_descs(p, b):
            pltpu.make_async_copy(*d).wait()

    def pos_compute(kg, s, p):
        base = s * _L

        def pos_body(t, c):
            rows = t * 16 + lane
            msk = rows < _L
            for tbl, pidx, cbase in ((p1_v, p_idx[0][kg], 0),
                                     (p2_v, p_idx[1][kg], _POS_DIM)):
                iv = pidx[pl.ds(base + t * 16, 16)]
                iv = jnp.where(msk, iv * _POS_DIM, 0)
                for j in range(_POS_DIM):
                    colv = jnp.full((16,), j, jnp.int32)
                    vals = plsc.load_gather(tbl, [iv + j])
                    plsc.store_scatter(pos_sc, [rows, colv + cbase],
                                       vals, mask=msk)
            return c

        lax.fori_loop(0, (_L + 15) // 16, pos_body, 0)

    def e12_fill(kg, s, p):
        evs = ([e_rows[kg][s, pl.ds(16 * v, 16)] for v in range(8)] +
               [e_rows[kg][_GRP + s, pl.ds(16 * v, 16)] for v in range(8)])

        def brow(r, c):
            for v in range(16):
                e12[p][r, pl.ds(16 * v, 16)] = evs[v]
            return c

        lax.fori_loop(0, _EROWS, brow, 0, unroll=4)

    def sentence(i_w, kg, s, p, b):
        @pl.when(i_w >= 2)
        def _():
            drain_other(p, b)

        drain_gather(p)

        @pl.when(i_w >= 1)
        def _():
            pltpu.make_async_copy(*pos_desc(b)).wait()

        pos_compute(kg, s, p)
        e12_fill(kg, s, p)
        issue_scatters(p, b)

        @pl.when(i_w >= 1)
        def _():
            drain_word(1 - p, b)

    issue_grp(0, 0, b0)
    pltpu.sync_copy(p1w, p1_v)
    pltpu.sync_copy(p2w, p2_v)

    def run_group(sg, gi):
        g = 2 * sg + gi
        bg = b0 + g * _GRP
        drain_grp(gi, g)
        if gi == 0:
            issue_grp(1, g + 1, bg + _GRP)
        else:
            @pl.when(sg <= _NGRP // 2 - 2)
            def _():
                issue_grp(0, g + 1, bg + _GRP)
        issue_gather(gi, 0, 0)

        def pair(j2, c):
            i_w = g * _GRP + 2 * j2
            b = bg + 2 * j2
            s = 2 * j2
            sentence(i_w, gi, s, 0, b)
            issue_gather(gi, s + 1, 1)
            sentence(i_w + 1, gi, s + 1, 1, b + 1)

            @pl.when(j2 <= 2)
            def _():
                issue_gather(gi, s + 2, 0)
            return c

        lax.fori_loop(0, _GRP // 2, pair, 0)

    def sg_body(sg, c):
        run_group(sg, 0)
        run_group(sg, 1)
        return c

    lax.fori_loop(0, _NGRP // 2, sg_body, 0)

    drain_other(0, b0)
    drain_other(1, b0)
    drain_word(1, b0)
    pltpu.make_async_copy(*pos_desc(b0)).wait()


def _run(X, XP1, XP2, XE1, XE2, table, p1w, p2w):
    mesh = plsc.VectorSubcoreMesh(core_axis_name="c", subcore_axis_name="s")
    f = pl.kernel(
        _ea_body,
        mesh=mesh,
        compiler_params=pltpu.CompilerParams(needs_layout_passes=False),
        out_type=(
            jax.ShapeDtypeStruct((_B, _L, _D + 2 * _POS_DIM), jnp.float32),
            jax.ShapeDtypeStruct((_B, _L, 3 * _D), jnp.float32),
        ),
        scratch_types=[
            pltpu.VMEM((_GTOK + 8,), jnp.int32),
            pltpu.VMEM((_GTOK + 8,), jnp.int32),
            pltpu.VMEM((_GTOK + 8,), jnp.int32),
            pltpu.VMEM((_GTOK + 8,), jnp.int32),
            pltpu.VMEM((_GTOK + 8,), jnp.int32),
            pltpu.VMEM((_GTOK + 8,), jnp.int32),
            pltpu.VMEM((2 * _GRP,), jnp.int32),
            pltpu.VMEM((2 * _GRP,), jnp.int32),
            pltpu.VMEM((2 * _GRP, _D), jnp.float32),
            pltpu.VMEM((2 * _GRP, _D), jnp.float32),
            pltpu.VMEM((_L, _D), jnp.float32),
            pltpu.VMEM((_L, _D), jnp.float32),
            pltpu.VMEM((_EROWS, 2 * _D), jnp.float32),
            pltpu.VMEM((_EROWS, 2 * _D), jnp.float32),
            pltpu.VMEM((_L, 2 * _POS_DIM), jnp.float32),
            pltpu.VMEM((1024,), jnp.float32),
            pltpu.VMEM((1024,), jnp.float32),
            pltpu.SemaphoreType.DMA,
            pltpu.SemaphoreType.DMA,
            pltpu.SemaphoreType.DMA,
            pltpu.SemaphoreType.DMA,
            pltpu.SemaphoreType.DMA,
            pltpu.SemaphoreType.DMA,
            pltpu.SemaphoreType.DMA,
            pltpu.SemaphoreType.DMA,
            pltpu.SemaphoreType.DMA,
        ],
    )
    return f(X, XP1, XP2, XE1, XE2, table, p1w, p2w)


_run = jax.jit(_run)


def kernel(X, X_Pos1, X_Pos2, X_Ent1, X_Ent2, word_embedding, pos1_weight, pos2_weight):
    p1f = jnp.pad(pos1_weight.reshape(-1), (0, 1024 - _POS_VOCAB * _POS_DIM))
    p2f = jnp.pad(pos2_weight.reshape(-1), (0, 1024 - _POS_VOCAB * _POS_DIM))
    return _run(X.reshape(-1), X_Pos1.reshape(-1), X_Pos2.reshape(-1),
                X_Ent1, X_Ent2, word_embedding, p1f, p2f)


# speedup vs baseline: 6.9681x; 1.0038x over previous
import jax
import jax.numpy as jnp
from jax import lax
from jax.experimental import pallas as pl
from jax.experimental.pallas import tpu as pltpu
from jax.experimental.pallas import tpu_sc as plsc

_VOCAB = 100000
_D = 128
_POS_DIM = 5
_POS_VOCAB = 201
_B = 4096
_L = 200
_NC = 2
_NS = 16
_NW = _NC * _NS
_BPW = _B // _NW
_GRP = 8
_NGRP = _BPW // _GRP
_GTOK = _GRP * _L
_EROWS = 64


def _ea_body(X, XP1, XP2, XE1, XE2, table, p1w, p2w, xp_out, xe_out,
             x_idx0, x_idx1, p1_idx0, p1_idx1, p2_idx0, p2_idx1,
             e_idx0, e_idx1, e_rows0, e_rows1, word0, word1,
             e12a, e12b, pos_sc, p1_v, p2_v,
             sem_g0, sem_g1, sem_w0, sem_w1, sem_o0, sem_o1,
             sem_r0, sem_r1, sem_p):
    sem_g = (sem_g0, sem_g1)
    sem_w = (sem_w0, sem_w1)
    sem_o = (sem_o0, sem_o1)
    sem_r = (sem_r0, sem_r1)
    x_idx = (x_idx0, x_idx1)
    p_idx = ((p1_idx0, p1_idx1), (p2_idx0, p2_idx1))
    e_idx = (e_idx0, e_idx1)
    e_rows = (e_rows0, e_rows1)
    word = (word0, word1)
    e12 = (e12a, e12b)

    wid = lax.axis_index("s") * _NC + lax.axis_index("c")
    b0 = wid * _BPW
    lane = lax.iota(jnp.int32, 16)

    def grp_descs(kg, g):
        tok0 = (b0 + g * _GRP) * _L
        return (
            (table.at[e_idx[kg].at[pl.ds(0, _GRP)]], e_rows[kg].at[pl.ds(0, _GRP)], sem_r[kg]),
            (table.at[e_idx[kg].at[pl.ds(_GRP, _GRP)]], e_rows[kg].at[pl.ds(_GRP, _GRP)], sem_r[kg]),
            (X.at[pl.ds(tok0, _GTOK)], x_idx[kg].at[pl.ds(0, _GTOK)], sem_r[kg]),
            (XP1.at[pl.ds(tok0, _GTOK)], p_idx[0][kg].at[pl.ds(0, _GTOK)], sem_r[kg]),
            (XP2.at[pl.ds(tok0, _GTOK)], p_idx[1][kg].at[pl.ds(0, _GTOK)], sem_r[kg]),
        )

    def grp_drain_descs(kg, g):
        tok0 = (b0 + g * _GRP) * _L
        return (
            (table.at[pl.ds(0, _GRP)], e_rows[kg].at[pl.ds(0, _GRP)], sem_r[kg]),
            (table.at[pl.ds(0, _GRP)], e_rows[kg].at[pl.ds(_GRP, _GRP)], sem_r[kg]),
            (X.at[pl.ds(tok0, _GTOK)], x_idx[kg].at[pl.ds(0, _GTOK)], sem_r[kg]),
            (XP1.at[pl.ds(tok0, _GTOK)], p_idx[0][kg].at[pl.ds(0, _GTOK)], sem_r[kg]),
            (XP2.at[pl.ds(tok0, _GTOK)], p_idx[1][kg].at[pl.ds(0, _GTOK)], sem_r[kg]),
        )

    def issue_grp(kg, g, bg):
        pltpu.sync_copy(XE1.at[pl.ds(bg, _GRP)], e_idx[kg].at[pl.ds(0, _GRP)])
        pltpu.sync_copy(XE2.at[pl.ds(bg, _GRP)], e_idx[kg].at[pl.ds(_GRP, _GRP)])
        for d in grp_descs(kg, g):
            pltpu.async_copy(*d)

    def drain_grp(kg, g):
        for d in grp_drain_descs(kg, g):
            pltpu.make_async_copy(*d).wait()

    def issue_gather(kg, s, p):
        t0 = s * _L
        pltpu.async_copy(table.at[x_idx[kg].at[pl.ds(t0, 104)]],
                         word[p].at[pl.ds(0, 104)], sem_g[p])
        pltpu.async_copy(table.at[x_idx[kg].at[pl.ds(t0 + 104, 96)]],
                         word[p].at[pl.ds(104, 96)], sem_g[p])

    def drain_gather(p):
        pltpu.make_async_copy(table.at[pl.ds(0, 104)],
                              word[p].at[pl.ds(0, 104)], sem_g[p]).wait()
        pltpu.make_async_copy(table.at[pl.ds(0, 96)],
                              word[p].at[pl.ds(104, 96)], sem_g[p]).wait()

    def word_descs(p, b):
        return (
            (word[p], xp_out.at[b, :, pl.ds(0, _D)], sem_w[p]),
            (word[p], xe_out.at[b, :, pl.ds(0, _D)], sem_w[p]),
        )

    def other_descs(p, b):
        ds = []
        for k in range(3):
            ds.append((e12[p],
                       xe_out.at[b, pl.ds(_EROWS * k, _EROWS), pl.ds(_D, 2 * _D)],
                       sem_o[p]))
        ds.append((e12[p].at[pl.ds(0, 8)],
                   xe_out.at[b, pl.ds(192, 8), pl.ds(_D, 2 * _D)], sem_o[p]))
        return ds

    def pos_desc(b):
        return (pos_sc, xp_out.at[b, :, pl.ds(_D, 2 * _POS_DIM)], sem_p)

    def issue_word(p, b):
        for d in word_descs(p, b):
            pltpu.async_copy(*d)

    def issue_other(p, b):
        for d in tuple(other_descs(p, b)) + (pos_desc(b),):
            pltpu.async_copy(*d)

    def drain_word(p, b):
        for d in word_descs(p, b):
            pltpu.make_async_copy(*d).wait()

    def drain_other(p, b):
        for d in other_descs(p, b):
            pltpu.make_async_copy(*d).wait()

    def pos_compute(kg, s, p):
        base = s * _L

        def pos_body(t, c):
            rows = t * 16 + lane
            msk = rows < _L
            for tbl, pidx, cbase in ((p1_v, p_idx[0][kg], 0),
                                     (p2_v, p_idx[1][kg], _POS_DIM)):
                iv = pidx[pl.ds(base + t * 16, 16)]
                iv = jnp.where(msk, iv * _POS_DIM, 0)
                for j in range(_POS_DIM):
                    colv = jnp.full((16,), j, jnp.int32)
                    vals = plsc.load_gather(tbl, [iv + j])
                    plsc.store_scatter(pos_sc, [rows, colv + cbase],
                                       vals, mask=msk)
            return c

        lax.fori_loop(0, (_L + 15) // 16, pos_body, 0)

    def e12_fill(kg, s, p):
        evs = ([e_rows[kg][s, pl.ds(16 * v, 16)] for v in range(8)] +
               [e_rows[kg][_GRP + s, pl.ds(16 * v, 16)] for v in range(8)])

        def brow(r, c):
            for v in range(16):
                e12[p][r, pl.ds(16 * v, 16)] = evs[v]
            return c

        lax.fori_loop(0, _EROWS, brow, 0, unroll=4)

    def sentence(i_w, kg, s, p, b, nxt=None):
        drain_gather(p)
        issue_word(p, b)

        @pl.when(i_w >= 1)
        def _():
            drain_word(1 - p, b)

        if nxt is not None:
            nxt()

        @pl.when(i_w >= 2)
        def _():
            drain_other(p, b)

        e12_fill(kg, s, p)

        @pl.when(i_w >= 1)
        def _():
            pltpu.make_async_copy(*pos_desc(b)).wait()

        pos_compute(kg, s, p)
        issue_other(p, b)

    issue_grp(0, 0, b0)
    pltpu.sync_copy(p1w, p1_v)
    pltpu.sync_copy(p2w, p2_v)

    def run_group(sg, gi):
        g = 2 * sg + gi
        bg = b0 + g * _GRP
        drain_grp(gi, g)
        if gi == 0:
            issue_grp(1, g + 1, bg + _GRP)
        else:
            @pl.when(sg <= _NGRP // 2 - 2)
            def _():
                issue_grp(0, g + 1, bg + _GRP)
        issue_gather(gi, 0, 0)

        def pair(j2, c):
            i_w = g * _GRP + 2 * j2
            b = bg + 2 * j2
            s = 2 * j2
            sentence(i_w, gi, s, 0, b,
                     nxt=lambda: issue_gather(gi, s + 1, 1))

            def nxt1():
                @pl.when(j2 <= 2)
                def _():
                    issue_gather(gi, s + 2, 0)

            sentence(i_w + 1, gi, s + 1, 1, b + 1, nxt=nxt1)
            return c

        lax.fori_loop(0, _GRP // 2, pair, 0)

    def sg_body(sg, c):
        run_group(sg, 0)
        run_group(sg, 1)
        return c

    lax.fori_loop(0, _NGRP // 2, sg_body, 0)

    drain_other(0, b0)
    drain_other(1, b0)
    drain_word(1, b0)
    pltpu.make_async_copy(*pos_desc(b0)).wait()


def _run(X, XP1, XP2, XE1, XE2, table, p1w, p2w):
    mesh = plsc.VectorSubcoreMesh(core_axis_name="c", subcore_axis_name="s")
    f = pl.kernel(
        _ea_body,
        mesh=mesh,
        compiler_params=pltpu.CompilerParams(needs_layout_passes=False),
        out_type=(
            jax.ShapeDtypeStruct((_B, _L, _D + 2 * _POS_DIM), jnp.float32),
            jax.ShapeDtypeStruct((_B, _L, 3 * _D), jnp.float32),
        ),
        scratch_types=[
            pltpu.VMEM((_GTOK + 8,), jnp.int32),
            pltpu.VMEM((_GTOK + 8,), jnp.int32),
            pltpu.VMEM((_GTOK + 8,), jnp.int32),
            pltpu.VMEM((_GTOK + 8,), jnp.int32),
            pltpu.VMEM((_GTOK + 8,), jnp.int32),
            pltpu.VMEM((_GTOK + 8,), jnp.int32),
            pltpu.VMEM((2 * _GRP,), jnp.int32),
            pltpu.VMEM((2 * _GRP,), jnp.int32),
            pltpu.VMEM((2 * _GRP, _D), jnp.float32),
            pltpu.VMEM((2 * _GRP, _D), jnp.float32),
            pltpu.VMEM((_L, _D), jnp.float32),
            pltpu.VMEM((_L, _D), jnp.float32),
            pltpu.VMEM((_EROWS, 2 * _D), jnp.float32),
            pltpu.VMEM((_EROWS, 2 * _D), jnp.float32),
            pltpu.VMEM((_L, 2 * _POS_DIM), jnp.float32),
            pltpu.VMEM((1024,), jnp.float32),
            pltpu.VMEM((1024,), jnp.float32),
            pltpu.SemaphoreType.DMA,
            pltpu.SemaphoreType.DMA,
            pltpu.SemaphoreType.DMA,
            pltpu.SemaphoreType.DMA,
            pltpu.SemaphoreType.DMA,
            pltpu.SemaphoreType.DMA,
            pltpu.SemaphoreType.DMA,
            pltpu.SemaphoreType.DMA,
            pltpu.SemaphoreType.DMA,
        ],
    )
    return f(X, XP1, XP2, XE1, XE2, table, p1w, p2w)


_run = jax.jit(_run)


def kernel(X, X_Pos1, X_Pos2, X_Ent1, X_Ent2, word_embedding, pos1_weight, pos2_weight):
    p1f = jnp.pad(pos1_weight.reshape(-1), (0, 1024 - _POS_VOCAB * _POS_DIM))
    p2f = jnp.pad(pos2_weight.reshape(-1), (0, 1024 - _POS_VOCAB * _POS_DIM))
    return _run(X.reshape(-1), X_Pos1.reshape(-1), X_Pos2.reshape(-1),
                X_Ent1, X_Ent2, word_embedding, p1f, p2f)


# speedup vs baseline: 7.3269x; 1.0515x over previous
import jax
import jax.numpy as jnp
from jax import lax
from jax.experimental import pallas as pl
from jax.experimental.pallas import tpu as pltpu
from jax.experimental.pallas import tpu_sc as plsc

_VOCAB = 100000
_D = 128
_POS_DIM = 5
_POS_VOCAB = 201
_B = 4096
_L = 200
_NC = 2
_NS = 16
_NW = _NC * _NS
_BPW = _B // _NW
_GRP = 8
_NGRP = _BPW // _GRP
_GTOK = _GRP * _L
_EROWS = 64


def _xp_body(X, XP1, XP2, table, p1w, p2w, xp_out,
             x_idx0, x_idx1, p1_idx0, p1_idx1, p2_idx0, p2_idx1,
             word0, word1, pos_sc, p1_v, p2_v,
             sem_g0, sem_g1, sem_w0, sem_w1, sem_r0, sem_r1, sem_p):
    sem_g = (sem_g0, sem_g1)
    sem_w = (sem_w0, sem_w1)
    sem_r = (sem_r0, sem_r1)
    x_idx = (x_idx0, x_idx1)
    p_idx = ((p1_idx0, p1_idx1), (p2_idx0, p2_idx1))
    word = (word0, word1)

    wid = lax.axis_index("s") * _NC + lax.axis_index("c")
    b0 = wid * _BPW
    lane = lax.iota(jnp.int32, 16)

    def grp_descs(kg, g):
        tok0 = (b0 + g * _GRP) * _L
        return (
            (X.at[pl.ds(tok0, _GTOK)], x_idx[kg].at[pl.ds(0, _GTOK)], sem_r[kg]),
            (XP1.at[pl.ds(tok0, _GTOK)], p_idx[0][kg].at[pl.ds(0, _GTOK)], sem_r[kg]),
            (XP2.at[pl.ds(tok0, _GTOK)], p_idx[1][kg].at[pl.ds(0, _GTOK)], sem_r[kg]),
        )

    def issue_grp(kg, g):
        for d in grp_descs(kg, g):
            pltpu.async_copy(*d)

    def drain_grp(kg, g):
        for d in grp_descs(kg, g):
            pltpu.make_async_copy(*d).wait()

    def issue_gather(kg, s, p):
        t0 = s * _L
        pltpu.async_copy(table.at[x_idx[kg].at[pl.ds(t0, 104)]],
                         word[p].at[pl.ds(0, 104)], sem_g[p])
        pltpu.async_copy(table.at[x_idx[kg].at[pl.ds(t0 + 104, 96)]],
                         word[p].at[pl.ds(104, 96)], sem_g[p])

    def drain_gather(p):
        pltpu.make_async_copy(table.at[pl.ds(0, 104)],
                              word[p].at[pl.ds(0, 104)], sem_g[p]).wait()
        pltpu.make_async_copy(table.at[pl.ds(0, 96)],
                              word[p].at[pl.ds(104, 96)], sem_g[p]).wait()

    def word_desc(p, b):
        return (word[p], xp_out.at[b, :, pl.ds(0, _D)], sem_w[p])

    def pos_desc(b):
        return (pos_sc, xp_out.at[b, :, pl.ds(_D, 2 * _POS_DIM)], sem_p)

    def pos_compute(kg, s):
        base = s * _L

        def pos_body(t, c):
            rows = t * 16 + lane
            msk = rows < _L
            for tbl, pidx, cbase in ((p1_v, p_idx[0][kg], 0),
                                     (p2_v, p_idx[1][kg], _POS_DIM)):
                iv = pidx[pl.ds(base + t * 16, 16)]
                iv = jnp.where(msk, iv * _POS_DIM, 0)
                for j in range(_POS_DIM):
                    colv = jnp.full((16,), j, jnp.int32)
                    vals = plsc.load_gather(tbl, [iv + j])
                    plsc.store_scatter(pos_sc, [rows, colv + cbase],
                                       vals, mask=msk)
            return c

        lax.fori_loop(0, (_L + 15) // 16, pos_body, 0)

    def sentence(i_w, kg, s, p, b, nxt=None):
        drain_gather(p)
        pltpu.async_copy(*word_desc(p, b))

        @pl.when(i_w >= 1)
        def _():
            pltpu.make_async_copy(*word_desc(1 - p, b)).wait()

        if nxt is not None:
            nxt()

        @pl.when(i_w >= 1)
        def _():
            pltpu.make_async_copy(*pos_desc(b)).wait()

        pos_compute(kg, s)
        pltpu.async_copy(*pos_desc(b))

    pltpu.sync_copy(p1w, p1_v)
    pltpu.sync_copy(p2w, p2_v)
    issue_grp(0, 0)

    def run_group(sg, gi):
        g = 2 * sg + gi
        bg = b0 + g * _GRP
        drain_grp(gi, g)
        if gi == 0:
            issue_grp(1, g + 1)
        else:
            @pl.when(sg <= _NGRP // 2 - 2)
            def _():
                issue_grp(0, g + 1)
        issue_gather(gi, 0, 0)

        def pair(j2, c):
            i_w = g * _GRP + 2 * j2
            b = bg + 2 * j2
            s = 2 * j2
            sentence(i_w, gi, s, 0, b, nxt=lambda: issue_gather(gi, s + 1, 1))

            def nxt1():
                @pl.when(j2 <= 2)
                def _():
                    issue_gather(gi, s + 2, 0)

            sentence(i_w + 1, gi, s + 1, 1, b + 1, nxt=nxt1)
            return c

        lax.fori_loop(0, _GRP // 2, pair, 0)

    def sg_body(sg, c):
        run_group(sg, 0)
        run_group(sg, 1)
        return c

    lax.fori_loop(0, _NGRP // 2, sg_body, 0)

    pltpu.make_async_copy(*word_desc(1, b0)).wait()
    pltpu.make_async_copy(*pos_desc(b0)).wait()


def _xe_body(XE1, XE2, table, xp_in, xe_out,
             e_idx0, e_idx1, e_rows0, e_rows1, word0, word1, e12a, e12b,
             sem_g0, sem_g1, sem_w0, sem_w1, sem_o0, sem_o1,
             sem_r0, sem_r1):
    sem_g = (sem_g0, sem_g1)
    sem_w = (sem_w0, sem_w1)
    sem_o = (sem_o0, sem_o1)
    sem_r = (sem_r0, sem_r1)
    e_idx = (e_idx0, e_idx1)
    e_rows = (e_rows0, e_rows1)
    word = (word0, word1)
    e12 = (e12a, e12b)

    wid = lax.axis_index("s") * _NC + lax.axis_index("c")
    b0 = wid * _BPW

    def grp_descs(kg):
        return (
            (table.at[e_idx[kg].at[pl.ds(0, _GRP)]],
             e_rows[kg].at[pl.ds(0, _GRP)], sem_r[kg]),
            (table.at[e_idx[kg].at[pl.ds(_GRP, _GRP)]],
             e_rows[kg].at[pl.ds(_GRP, _GRP)], sem_r[kg]),
        )

    def grp_drain_descs(kg):
        return (
            (table.at[pl.ds(0, _GRP)], e_rows[kg].at[pl.ds(0, _GRP)], sem_r[kg]),
            (table.at[pl.ds(0, _GRP)], e_rows[kg].at[pl.ds(_GRP, _GRP)], sem_r[kg]),
        )

    def issue_grp(kg, bg):
        pltpu.sync_copy(XE1.at[pl.ds(bg, _GRP)], e_idx[kg].at[pl.ds(0, _GRP)])
        pltpu.sync_copy(XE2.at[pl.ds(bg, _GRP)], e_idx[kg].at[pl.ds(_GRP, _GRP)])
        for d in grp_descs(kg):
            pltpu.async_copy(*d)

    def drain_grp(kg):
        for d in grp_drain_descs(kg):
            pltpu.make_async_copy(*d).wait()

    def read_desc(p, b):
        return (xp_in.at[b, :, pl.ds(0, _D)], word[p], sem_g[p])

    def word_desc(p, b):
        return (word[p], xe_out.at[b, :, pl.ds(0, _D)], sem_w[p])

    def other_descs(p, b):
        ds = []
        for k in range(3):
            ds.append((e12[p],
                       xe_out.at[b, pl.ds(_EROWS * k, _EROWS), pl.ds(_D, 2 * _D)],
                       sem_o[p]))
        ds.append((e12[p].at[pl.ds(0, 8)],
                   xe_out.at[b, pl.ds(192, 8), pl.ds(_D, 2 * _D)], sem_o[p]))
        return ds

    def e12_fill(kg, s, p):
        evs = ([e_rows[kg][s, pl.ds(16 * v, 16)] for v in range(8)] +
               [e_rows[kg][_GRP + s, pl.ds(16 * v, 16)] for v in range(8)])

        def brow(r, c):
            for v in range(16):
                e12[p][r, pl.ds(16 * v, 16)] = evs[v]
            return c

        lax.fori_loop(0, _EROWS, brow, 0, unroll=4)

    def sentence(i_w, kg, s, p, b, nxt=None):
        pltpu.make_async_copy(*read_desc(p, b)).wait()
        pltpu.async_copy(*word_desc(p, b))

        @pl.when(i_w >= 1)
        def _():
            pltpu.make_async_copy(*word_desc(1 - p, b)).wait()

        if nxt is not None:
            nxt()

        @pl.when(i_w >= 2)
        def _():
            for d in other_descs(p, b):
                pltpu.make_async_copy(*d).wait()

        e12_fill(kg, s, p)
        for d in other_descs(p, b):
            pltpu.async_copy(*d)

    issue_grp(0, b0)
    pltpu.async_copy(*read_desc(0, b0))

    def run_group(sg, gi):
        g = 2 * sg + gi
        bg = b0 + g * _GRP
        drain_grp(gi)
        if gi == 0:
            issue_grp(1, bg + _GRP)
        else:
            @pl.when(sg <= _NGRP // 2 - 2)
            def _():
                issue_grp(0, bg + _GRP)

        def pair(j2, c):
            i_w = g * _GRP + 2 * j2
            b = bg + 2 * j2
            s = 2 * j2
            sentence(i_w, gi, s, 0, b,
                     nxt=lambda: pltpu.async_copy(*read_desc(1, b + 1)))

            def nxt1():
                @pl.when(i_w + 2 <= _BPW - 1)
                def _():
                    pltpu.async_copy(*read_desc(0, b + 2))

            sentence(i_w + 1, gi, s + 1, 1, b + 1, nxt=nxt1)
            return c

        lax.fori_loop(0, _GRP // 2, pair, 0)

    def sg_body(sg, c):
        run_group(sg, 0)
        run_group(sg, 1)
        return c

    lax.fori_loop(0, _NGRP // 2, sg_body, 0)

    for d in other_descs(0, b0):
        pltpu.make_async_copy(*d).wait()
    for d in other_descs(1, b0):
        pltpu.make_async_copy(*d).wait()
    pltpu.make_async_copy(*word_desc(1, b0)).wait()


def _run(X, XP1, XP2, XE1, XE2, table, p1w, p2w):
    mesh = plsc.VectorSubcoreMesh(core_axis_name="c", subcore_axis_name="s")
    xp_k = pl.kernel(
        _xp_body,
        mesh=mesh,
        compiler_params=pltpu.CompilerParams(needs_layout_passes=False),
        out_type=jax.ShapeDtypeStruct((_B, _L, _D + 2 * _POS_DIM), jnp.float32),
        scratch_types=[
            pltpu.VMEM((_GTOK + 8,), jnp.int32),
            pltpu.VMEM((_GTOK + 8,), jnp.int32),
            pltpu.VMEM((_GTOK + 8,), jnp.int32),
            pltpu.VMEM((_GTOK + 8,), jnp.int32),
            pltpu.VMEM((_GTOK + 8,), jnp.int32),
            pltpu.VMEM((_GTOK + 8,), jnp.int32),
            pltpu.VMEM((_L, _D), jnp.float32),
            pltpu.VMEM((_L, _D), jnp.float32),
            pltpu.VMEM((_L, 2 * _POS_DIM), jnp.float32),
            pltpu.VMEM((1024,), jnp.float32),
            pltpu.VMEM((1024,), jnp.float32),
        ] + [pltpu.SemaphoreType.DMA] * 7,
    )
    xe_k = pl.kernel(
        _xe_body,
        mesh=mesh,
        compiler_params=pltpu.CompilerParams(needs_layout_passes=False),
        out_type=jax.ShapeDtypeStruct((_B, _L, 3 * _D), jnp.float32),
        scratch_types=[
            pltpu.VMEM((2 * _GRP,), jnp.int32),
            pltpu.VMEM((2 * _GRP,), jnp.int32),
            pltpu.VMEM((2 * _GRP, _D), jnp.float32),
            pltpu.VMEM((2 * _GRP, _D), jnp.float32),
            pltpu.VMEM((_L, _D), jnp.float32),
            pltpu.VMEM((_L, _D), jnp.float32),
            pltpu.VMEM((_EROWS, 2 * _D), jnp.float32),
            pltpu.VMEM((_EROWS, 2 * _D), jnp.float32),
        ] + [pltpu.SemaphoreType.DMA] * 8,
    )
    xp = xp_k(X, XP1, XP2, table, p1w, p2w)
    xe = xe_k(XE1, XE2, table, xp)
    return xp, xe


_run = jax.jit(_run)


def kernel(X, X_Pos1, X_Pos2, X_Ent1, X_Ent2, word_embedding, pos1_weight, pos2_weight):
    p1f = jnp.pad(pos1_weight.reshape(-1), (0, 1024 - _POS_VOCAB * _POS_DIM))
    p2f = jnp.pad(pos2_weight.reshape(-1), (0, 1024 - _POS_VOCAB * _POS_DIM))
    return _run(X.reshape(-1), X_Pos1.reshape(-1), X_Pos2.reshape(-1),
                X_Ent1, X_Ent2, word_embedding, p1f, p2f)
